# baseline (speedup 1.0000x reference)
import functools

import jax
import jax.numpy as jnp
from jax import lax
from jax.experimental import pallas as pl
from jax.experimental.pallas import tpu as pltpu
from jax.experimental.pallas import tpu_sc as plsc


def _sc_transe(B, D, n_workers):
    b_per_w = B // n_workers
    gsz = 16
    n_groups = b_per_w // gsz
    rows = gsz
    mesh = plsc.VectorSubcoreMesh(core_axis_name="c", subcore_axis_name="s")
    num_cores = 2

    @functools.partial(
        pl.kernel,
        mesh=mesh,
        out_type=jax.ShapeDtypeStruct((B,), jnp.float32),
        compiler_params=pltpu.CompilerParams(
            needs_layout_passes=False, use_tc_tiling_on_sc=True),
        scratch_types=[
            pltpu.VMEM((b_per_w,), jnp.int32),
            pltpu.VMEM((b_per_w,), jnp.int32),
            pltpu.VMEM((b_per_w,), jnp.int32),
            pltpu.VMEM((rows, D), jnp.float32),
            pltpu.VMEM((rows, D), jnp.float32),
            pltpu.VMEM((rows, D), jnp.float32),
            pltpu.VMEM((rows, D), jnp.float32),
            pltpu.VMEM((rows, D), jnp.float32),
            pltpu.VMEM((rows, D), jnp.float32),
            pltpu.VMEM((rows, D), jnp.float32),
            pltpu.VMEM((rows, D), jnp.float32),
            pltpu.VMEM((rows, D), jnp.float32),
            pltpu.VMEM((b_per_w,), jnp.float32),
            pltpu.SemaphoreType.DMA,
            pltpu.SemaphoreType.DMA,
            pltpu.SemaphoreType.DMA,
        ],
    )
    def k(heads_hbm, rel_hbm, tails_hbm, ev_hbm, rv_hbm, out_hbm,
          idx_h, idx_r, idx_t, e1a, era, e2a, e1b, erb, e2b,
          e1c, erc, e2c, outv, sem_a, sem_b, sem_c):
        wid = lax.axis_index("s") * num_cores + lax.axis_index("c")
        base = wid * b_per_w

        pltpu.sync_copy(heads_hbm.at[pl.ds(base, b_per_w)], idx_h)
        pltpu.sync_copy(rel_hbm.at[pl.ds(base, b_per_w)], idx_r)
        pltpu.sync_copy(tails_hbm.at[pl.ds(base, b_per_w)], idx_t)

        lane = lax.iota(jnp.int32, 16)

        def fire(g, e1m, erm, e2m, sem):
            off = g * gsz
            jh = idx_h[pl.ds(off, 16)]
            jr = idx_r[pl.ds(off, 16)]
            jt = idx_t[pl.ds(off, 16)]
            for l in range(16):
                dst = pl.ds(l, 1)
                pltpu.async_copy(ev_hbm.at[pl.ds(jh[l], 1), :], e1m.at[dst], sem)
                pltpu.async_copy(rv_hbm.at[pl.ds(jr[l], 1), :], erm.at[dst], sem)
                pltpu.async_copy(ev_hbm.at[pl.ds(jt[l], 1), :], e2m.at[dst], sem)

        def drain(e1m, erm, e2m, sem):
            src = ev_hbm.at[pl.ds(0, rows), :]
            pltpu.make_async_copy(src, e1m, sem).wait()
            pltpu.make_async_copy(src, erm, sem).wait()
            pltpu.make_async_copy(src, e2m, sem).wait()

        def compute(g, e1m, erm, e2m):
            off = g * gsz
            rh = rr = rt = lane
            acc = jnp.zeros((16,), jnp.float32)
            for c in range(D):
                col = jnp.full((16,), c, jnp.int32)
                h = plsc.load_gather(e1m, [rh, col])
                r = plsc.load_gather(erm, [rr, col])
                t = plsc.load_gather(e2m, [rt, col])
                d = (h + r) - t
                acc = acc + d * d
            outv[pl.ds(off, 16)] = -acc

        fire(0, e1a, era, e2a, sem_a)
        fire(1, e1b, erb, e2b, sem_b)

        def body(i, carry):
            g = i * 3
            fire(g + 2, e1c, erc, e2c, sem_c)
            drain(e1a, era, e2a, sem_a)
            compute(g, e1a, era, e2a)
            fire(g + 3, e1a, era, e2a, sem_a)
            drain(e1b, erb, e2b, sem_b)
            compute(g + 1, e1b, erb, e2b)
            fire(g + 4, e1b, erb, e2b, sem_b)
            drain(e1c, erc, e2c, sem_c)
            compute(g + 2, e1c, erc, e2c)
            return carry

        lax.fori_loop(0, (n_groups - 2) // 3, body, 0)
        drain(e1a, era, e2a, sem_a)
        compute(n_groups - 2, e1a, era, e2a)
        drain(e1b, erb, e2b, sem_b)
        compute(n_groups - 1, e1b, erb, e2b)

        pltpu.sync_copy(outv, out_hbm.at[pl.ds(base, b_per_w)])

    return k


def kernel(heads, relations, tails, entity_embedding, relation_embedding):
    B = heads.shape[0]
    D = entity_embedding.shape[1]
    k = _sc_transe(B, D, 32)
    return k(heads, relations, tails, entity_embedding, relation_embedding)
